# baseline (speedup 1.0000x reference)
import functools

import jax
import jax.numpy as jnp
from jax import lax
from jax.experimental import pallas as pl
from jax.experimental.pallas import tpu as pltpu
from jax.experimental.pallas import tpu_sc as plsc

_T = 2048
_DM = 2048
_RQ = 1536
_H = 32
_D = 128
_R = 64
_TOPK = 512
_EPS = 1e-6
_NW = 32
_GROUP = 512


def _rope_apply(x, c128, s128, nheads):
    if nheads > 1:
        c = jnp.concatenate([c128] * nheads, axis=-1)
        s = jnp.concatenate([s128] * nheads, axis=-1)
    else:
        c, s = c128, s128
    lane = lax.broadcasted_iota(jnp.int32, x.shape, 1) % 128
    swapped = jnp.where(lane < 32, jnp.roll(x, -32, axis=1), jnp.roll(x, 32, axis=1))
    return x * c + swapped * s


def _make_logits_kernel(row0, ncols):
    def _logits_kernel(q_ref, k_ref, hs_ref, wproj_ref, c_ref, s_ref, ck_ref,
                       sk_ref, out_ref):
        bt = q_ref.shape[0]
        t0 = row0 + pl.program_id(0) * bt
        qr = _rope_apply(q_ref[...], c_ref[...], s_ref[...], _H)
        kr = _rope_apply(k_ref[...], ck_ref[...], sk_ref[...], 1)
        scale = (float(_D) ** -0.5) * (float(_H) ** -0.5)
        w = jax.lax.dot_general(hs_ref[...], wproj_ref[...], (((1,), (0,)), ((), ())),
                                preferred_element_type=jnp.float32) * scale
        acc = jnp.zeros((bt, ncols), dtype=jnp.float32)
        for h in range(_H):
            qh = qr[:, h * _D:(h + 1) * _D]
            s = jax.lax.dot_general(qh, kr, (((1,), (1,)), ((), ())),
                                    preferred_element_type=jnp.float32)
            acc = acc + w[:, h:h + 1] * jnp.maximum(s, 0.0)
        row = t0 + lax.broadcasted_iota(jnp.int32, (bt, ncols), 0)
        col = lax.broadcasted_iota(jnp.int32, (bt, ncols), 1)
        b = lax.bitcast_convert_type(acc, jnp.int32)
        m = (b >> 31) | jnp.int32(-2147483648)
        key = lax.bitcast_convert_type(b ^ m, jnp.uint32)
        masked = (jnp.int32(_T - 1) - col).astype(jnp.uint32)
        out_ref[...] = jnp.where(row >= col, key, masked)

    return _logits_kernel


def _make_sc_topk(nch):
    rows_per = _GROUP // _NW

    def body(keys_hbm, out_hbm, ka, ia, kb, ib, sem):
        wid = lax.axis_index("s") * 2 + lax.axis_index("c")
        iota16 = lax.iota(jnp.int32, 16)

        def vs(buf, i):
            return buf[pl.ds(i * 16, 16)]

        def st(buf, i, x):
            buf[pl.ds(i * 16, 16)] = x

        def bitonic_reg(ks, vi):
            nv = len(ks)
            D = nv // 2
            while D >= 1:
                for blk in range(0, nv, 2 * D):
                    for j in range(D):
                        i0, i1 = blk + j, blk + j + D
                        cm = ks[i0] >= ks[i1]
                        hk = jnp.where(cm, ks[i0], ks[i1])
                        lk = jnp.where(cm, ks[i1], ks[i0])
                        hi = jnp.where(cm, vi[i0], vi[i1])
                        li = jnp.where(cm, vi[i1], vi[i0])
                        ks[i0], ks[i1] = hk, lk
                        vi[i0], vi[i1] = hi, li
                D //= 2
            for j in range(nv):
                ks[j], vi[j] = plsc.sort_key_val(ks[j], vi[j], descending=True)

        def merge_runs_reg(src_k, src_i, dst_k, dst_i, base, nv):
            ak = [vs(src_k, base + i) for i in range(nv)]
            ai = [vs(src_i, base + i) for i in range(nv)]
            bk = [lax.rev(vs(src_k, base + 2 * nv - 1 - i), (0,)) for i in range(nv)]
            bi = [lax.rev(vs(src_i, base + 2 * nv - 1 - i), (0,)) for i in range(nv)]
            uk, ui, vk_, vi_ = [], [], [], []
            for i in range(nv):
                cm = ak[i] >= bk[i]
                uk.append(jnp.where(cm, ak[i], bk[i]))
                ui.append(jnp.where(cm, ai[i], bi[i]))
                vk_.append(jnp.where(cm, bk[i], ak[i]))
                vi_.append(jnp.where(cm, bi[i], ai[i]))
            bitonic_reg(uk, ui)
            for i in range(nv):
                st(dst_k, base + i, uk[i])
                st(dst_i, base + i, ui[i])
            bitonic_reg(vk_, vi_)
            for i in range(nv):
                st(dst_k, base + nv + i, vk_[i])
                st(dst_i, base + nv + i, vi_[i])

        def merge_bitonic_mem(kbuf, ibuf, base, nv):
            D = nv // 2
            while D >= 4:
                for blk in range(0, nv, 2 * D):
                    for j in range(D):
                        i0 = base + blk + j
                        i1 = i0 + D
                        a_k, b_k = vs(kbuf, i0), vs(kbuf, i1)
                        a_i, b_i = vs(ibuf, i0), vs(ibuf, i1)
                        cm = a_k >= b_k
                        st(kbuf, i0, jnp.where(cm, a_k, b_k))
                        st(kbuf, i1, jnp.where(cm, b_k, a_k))
                        st(ibuf, i0, jnp.where(cm, a_i, b_i))
                        st(ibuf, i1, jnp.where(cm, b_i, a_i))
                D //= 2
            for blk in range(0, nv, 8):
                ks = [vs(kbuf, base + blk + j) for j in range(8)]
                vi = [vs(ibuf, base + blk + j) for j in range(8)]
                bitonic_reg(ks, vi)
                for j in range(8):
                    st(kbuf, base + blk + j, ks[j])
                    st(ibuf, base + blk + j, vi[j])

        def build_chunk(c, last):
            base32 = c * 32

            def sort16(i, _):
                kk, vv = plsc.sort_key_val(vs(ka, base32 + i),
                                           iota16 + (base32 + i) * 16,
                                           descending=True)
                st(ka, base32 + i, kk)
                st(ia, base32 + i, vv)
                return 0

            lax.fori_loop(0, 32, sort16, 0, unroll=4)
            bufs = ((ka, ia), (kb, ib))
            cur = 0
            for nv in (1, 2, 4, 8, 16):
                src_k, src_i = bufs[cur]
                dst_k, dst_i = bufs[1 - cur]
                npairs = 32 // (2 * nv)

                def pair_body(p, _, src_k=src_k, src_i=src_i, dst_k=dst_k,
                              dst_i=dst_i, nv=nv, base32=base32):
                    merge_runs_reg(src_k, src_i, dst_k, dst_i,
                                   base32 + p * 2 * nv, nv)
                    return 0

                lax.fori_loop(0, npairs, pair_body, 0)
                cur = 1 - cur
            if not last:
                for i in range(32):
                    st(ka, base32 + i, vs(kb, base32 + i))
                    st(ia, base32 + i, vs(ib, base32 + i))

        def trunc_merge_into_top(c):
            for i in range(32):
                a_k, a_i = vs(kb, i), vs(ib, i)
                rj = c * 32 + 31 - i
                b_k = lax.rev(vs(ka, rj), (0,))
                b_i = lax.rev(vs(ia, rj), (0,))
                cm = a_k >= b_k
                st(kb, i, jnp.where(cm, a_k, b_k))
                st(ib, i, jnp.where(cm, a_i, b_i))
            merge_bitonic_mem(kb, ib, 0, 32)

        def row_body(r, _):
            row = wid + _NW * r
            for c in range(nch):
                pltpu.sync_copy(keys_hbm.at[row, pl.ds(c * _TOPK, _TOPK)],
                                ka.at[pl.ds(c * _TOPK, _TOPK)])
                build_chunk(c, last=(c == nch - 1))
            if nch > 1:
                base32 = (nch - 1) * 32
                for i in range(32):
                    st(kb, i, vs(kb, base32 + i))
                    st(ib, i, vs(ib, base32 + i))
                for c in range(nch - 1):
                    trunc_merge_into_top(c)
            pltpu.sync_copy(ib.at[pl.ds(0, _TOPK)], out_hbm.at[row])
            return 0

        lax.fori_loop(0, rows_per, row_body, 0)

    return pl.kernel(
        body,
        mesh=plsc.VectorSubcoreMesh(core_axis_name="c", subcore_axis_name="s"),
        out_type=jax.ShapeDtypeStruct((_GROUP, _TOPK), jnp.int32),
        scratch_types=[
            pltpu.VMEM((_T,), jnp.uint32),
            pltpu.VMEM((_T,), jnp.int32),
            pltpu.VMEM((_T,), jnp.uint32),
            pltpu.VMEM((_T,), jnp.int32),
            pltpu.SemaphoreType.DMA,
        ],
        compiler_params=pltpu.CompilerParams(needs_layout_passes=False),
    )


_SC_TOPK = [_make_sc_topk(g + 1) for g in range(_T // _GROUP)]


def kernel(hidden_states, q_lora, positions, wq_b, wk, k_norm_w, k_norm_b, w_proj):
    q = jnp.matmul(q_lora, wq_b)
    k = jnp.matmul(hidden_states, wk)
    mu = jnp.mean(k, axis=-1, keepdims=True)
    var = jnp.mean((k - mu) ** 2, axis=-1, keepdims=True)
    k = (k - mu) / jnp.sqrt(var + _EPS) * k_norm_w + k_norm_b

    posf = positions.astype(jnp.float32)
    inv_freq = 1.0 / (10000.0 ** (jnp.arange(0, _R, 2, dtype=jnp.float32) / _R))
    ang = posf[:, None] * inv_freq[None, :]
    cos, sin = jnp.cos(ang), jnp.sin(ang)
    ones = jnp.ones((_T, 64), jnp.float32)
    zeros = jnp.zeros((_T, 64), jnp.float32)
    c128 = jnp.concatenate([cos, cos, ones], axis=-1)
    s128 = jnp.concatenate([-sin, sin, zeros], axis=-1)

    bt = 256
    ngroups = _T // _GROUP
    outs = []
    for g in range(ngroups):
        r0, r1 = g * _GROUP, (g + 1) * _GROUP
        ncols = (g + 1) * _GROUP
        keys_g = pl.pallas_call(
            _make_logits_kernel(r0, ncols),
            grid=(_GROUP // bt,),
            in_specs=[
                pl.BlockSpec((bt, _H * _D), lambda i: (i, 0)),
                pl.BlockSpec((ncols, _D), lambda i: (0, 0)),
                pl.BlockSpec((bt, _DM), lambda i: (i, 0)),
                pl.BlockSpec((_DM, _H), lambda i: (0, 0)),
                pl.BlockSpec((bt, _D), lambda i: (i, 0)),
                pl.BlockSpec((bt, _D), lambda i: (i, 0)),
                pl.BlockSpec((ncols, _D), lambda i: (0, 0)),
                pl.BlockSpec((ncols, _D), lambda i: (0, 0)),
            ],
            out_specs=pl.BlockSpec((bt, ncols), lambda i: (i, 0)),
            out_shape=jax.ShapeDtypeStruct((_GROUP, ncols), jnp.uint32),
        )(q[r0:r1], k, hidden_states[r0:r1], w_proj, c128[r0:r1], s128[r0:r1],
          c128, s128)
        outs.append(_SC_TOPK[g](keys_g))
    return jnp.concatenate(outs, axis=0)


# speedup vs baseline: 1.0199x; 1.0199x over previous
import functools

import jax
import jax.numpy as jnp
from jax import lax
from jax.experimental import pallas as pl
from jax.experimental.pallas import tpu as pltpu
from jax.experimental.pallas import tpu_sc as plsc

_T = 2048
_DM = 2048
_RQ = 1536
_H = 32
_D = 128
_R = 64
_TOPK = 512
_EPS = 1e-6
_NW = 32
_GROUP = 512


def _rope_apply(x, c128, s128, nheads):
    if nheads > 1:
        c = jnp.concatenate([c128] * nheads, axis=-1)
        s = jnp.concatenate([s128] * nheads, axis=-1)
    else:
        c, s = c128, s128
    lane = lax.broadcasted_iota(jnp.int32, x.shape, 1) % 128
    swapped = jnp.where(lane < 32, jnp.roll(x, -32, axis=1), jnp.roll(x, 32, axis=1))
    return x * c + swapped * s


def _make_logits_kernel(row0, ncols):
    def _logits_kernel(q_ref, k_ref, hs_ref, wproj_ref, c_ref, s_ref, ck_ref,
                       sk_ref, out_ref):
        bt = q_ref.shape[0]
        t0 = row0 + pl.program_id(0) * bt
        qr = _rope_apply(q_ref[...], c_ref[...], s_ref[...], _H)
        kr = _rope_apply(k_ref[...], ck_ref[...], sk_ref[...], 1)
        scale = (float(_D) ** -0.5) * (float(_H) ** -0.5)
        w = jax.lax.dot_general(hs_ref[...], wproj_ref[...], (((1,), (0,)), ((), ())),
                                preferred_element_type=jnp.float32) * scale
        acc = jnp.zeros((bt, ncols), dtype=jnp.float32)
        for h in range(_H):
            qh = qr[:, h * _D:(h + 1) * _D]
            s = jax.lax.dot_general(qh, kr, (((1,), (1,)), ((), ())),
                                    preferred_element_type=jnp.float32)
            acc = acc + w[:, h:h + 1] * jnp.maximum(s, 0.0)
        row = t0 + lax.broadcasted_iota(jnp.int32, (bt, ncols), 0)
        col = lax.broadcasted_iota(jnp.int32, (bt, ncols), 1)
        b = lax.bitcast_convert_type(acc, jnp.int32)
        m = (b >> 31) | jnp.int32(-2147483648)
        key = lax.bitcast_convert_type(b ^ m, jnp.uint32)
        masked = (jnp.int32(_T - 1) - col).astype(jnp.uint32)
        out_ref[...] = jnp.where(row >= col, key, masked)

    return _logits_kernel


def _make_sc_topk():
    def body(keys1, keys2, keys3, keys4, out_hbm, ka, ia, kb, ib, sem):
        wid = lax.axis_index("s") * 2 + lax.axis_index("c")
        iota16 = lax.iota(jnp.int32, 16)

        def vs(buf, i):
            return buf[pl.ds(i * 16, 16)]

        def st(buf, i, x):
            buf[pl.ds(i * 16, 16)] = x

        def bitonic_reg(ks, vi):
            nv = len(ks)
            D = nv // 2
            while D >= 1:
                for blk in range(0, nv, 2 * D):
                    for j in range(D):
                        i0, i1 = blk + j, blk + j + D
                        cm = ks[i0] >= ks[i1]
                        hk = jnp.where(cm, ks[i0], ks[i1])
                        lk = jnp.where(cm, ks[i1], ks[i0])
                        hi = jnp.where(cm, vi[i0], vi[i1])
                        li = jnp.where(cm, vi[i1], vi[i0])
                        ks[i0], ks[i1] = hk, lk
                        vi[i0], vi[i1] = hi, li
                D //= 2
            for j in range(nv):
                ks[j], vi[j] = plsc.sort_key_val(ks[j], vi[j], descending=True)

        def merge_runs_reg(src_k, src_i, dst_k, dst_i, base, nv):
            ak = [vs(src_k, base + i) for i in range(nv)]
            ai = [vs(src_i, base + i) for i in range(nv)]
            bk = [lax.rev(vs(src_k, base + 2 * nv - 1 - i), (0,)) for i in range(nv)]
            bi = [lax.rev(vs(src_i, base + 2 * nv - 1 - i), (0,)) for i in range(nv)]
            uk, ui, vk_, vi_ = [], [], [], []
            for i in range(nv):
                cm = ak[i] >= bk[i]
                uk.append(jnp.where(cm, ak[i], bk[i]))
                ui.append(jnp.where(cm, ai[i], bi[i]))
                vk_.append(jnp.where(cm, bk[i], ak[i]))
                vi_.append(jnp.where(cm, bi[i], ai[i]))
            bitonic_reg(uk, ui)
            for i in range(nv):
                st(dst_k, base + i, uk[i])
                st(dst_i, base + i, ui[i])
            bitonic_reg(vk_, vi_)
            for i in range(nv):
                st(dst_k, base + nv + i, vk_[i])
                st(dst_i, base + nv + i, vi_[i])

        def merge_bitonic_mem(kbuf, ibuf, base, nv):
            D = nv // 2
            while D >= 4:
                for blk in range(0, nv, 2 * D):
                    for j in range(D):
                        i0 = base + blk + j
                        i1 = i0 + D
                        a_k, b_k = vs(kbuf, i0), vs(kbuf, i1)
                        a_i, b_i = vs(ibuf, i0), vs(ibuf, i1)
                        cm = a_k >= b_k
                        st(kbuf, i0, jnp.where(cm, a_k, b_k))
                        st(kbuf, i1, jnp.where(cm, b_k, a_k))
                        st(ibuf, i0, jnp.where(cm, a_i, b_i))
                        st(ibuf, i1, jnp.where(cm, b_i, a_i))
                D //= 2
            for blk in range(0, nv, 8):
                ks = [vs(kbuf, base + blk + j) for j in range(8)]
                vi = [vs(ibuf, base + blk + j) for j in range(8)]
                bitonic_reg(ks, vi)
                for j in range(8):
                    st(kbuf, base + blk + j, ks[j])
                    st(ibuf, base + blk + j, vi[j])

        def build_chunk(c, last):
            base32 = c * 32

            def sort16(i, _):
                kk, vv = plsc.sort_key_val(vs(ka, base32 + i),
                                           iota16 + (base32 + i) * 16,
                                           descending=True)
                st(ka, base32 + i, kk)
                st(ia, base32 + i, vv)
                return 0

            lax.fori_loop(0, 32, sort16, 0, unroll=4)
            bufs = ((ka, ia), (kb, ib))
            cur = 0
            for nv in (1, 2, 4, 8, 16):
                src_k, src_i = bufs[cur]
                dst_k, dst_i = bufs[1 - cur]
                npairs = 32 // (2 * nv)

                def pair_body(p, _, src_k=src_k, src_i=src_i, dst_k=dst_k,
                              dst_i=dst_i, nv=nv, base32=base32):
                    merge_runs_reg(src_k, src_i, dst_k, dst_i,
                                   base32 + p * 2 * nv, nv)
                    return 0

                lax.fori_loop(0, npairs, pair_body, 0)
                cur = 1 - cur
            if not last:
                for i in range(32):
                    st(ka, base32 + i, vs(kb, base32 + i))
                    st(ia, base32 + i, vs(ib, base32 + i))

        def trunc_merge_into_top(c):
            for i in range(32):
                a_k, a_i = vs(kb, i), vs(ib, i)
                rj = c * 32 + 31 - i
                b_k = lax.rev(vs(ka, rj), (0,))
                b_i = lax.rev(vs(ia, rj), (0,))
                cm = a_k >= b_k
                st(kb, i, jnp.where(cm, a_k, b_k))
                st(ib, i, jnp.where(cm, a_i, b_i))
            merge_bitonic_mem(kb, ib, 0, 32)

        def make_row_body(keys_hbm, g):
            nch = g + 1

            def row_body(r, _):
                row_local = wid + _NW * r

                def chunk_body(c, _):
                    pltpu.sync_copy(keys_hbm.at[row_local, pl.ds(c * _TOPK, _TOPK)],
                                    ka.at[pl.ds(c * _TOPK, _TOPK)])
                    build_chunk(c, last=False)
                    return 0

                lax.fori_loop(0, nch, chunk_body, 0)
                for i in range(32):
                    st(kb, i, vs(ka, i))
                    st(ib, i, vs(ia, i))

                def merge_body(c, _):
                    trunc_merge_into_top(c)
                    return 0

                if nch > 1:
                    lax.fori_loop(1, nch, merge_body, 0)
                pltpu.sync_copy(ib.at[pl.ds(0, _TOPK)],
                                out_hbm.at[g * _GROUP + row_local])
                return 0

            return row_body

        for g, keys_hbm in enumerate((keys1, keys2, keys3, keys4)):
            lax.fori_loop(0, _GROUP // _NW, make_row_body(keys_hbm, g), 0)

    return pl.kernel(
        body,
        mesh=plsc.VectorSubcoreMesh(core_axis_name="c", subcore_axis_name="s"),
        out_type=jax.ShapeDtypeStruct((_T, _TOPK), jnp.int32),
        scratch_types=[
            pltpu.VMEM((_T,), jnp.uint32),
            pltpu.VMEM((_T,), jnp.int32),
            pltpu.VMEM((_T,), jnp.uint32),
            pltpu.VMEM((_T,), jnp.int32),
            pltpu.SemaphoreType.DMA,
        ],
        compiler_params=pltpu.CompilerParams(needs_layout_passes=False),
    )


_SC_TOPK = _make_sc_topk()


def kernel(hidden_states, q_lora, positions, wq_b, wk, k_norm_w, k_norm_b, w_proj):
    q = jnp.matmul(q_lora, wq_b)
    k = jnp.matmul(hidden_states, wk)
    mu = jnp.mean(k, axis=-1, keepdims=True)
    var = jnp.mean((k - mu) ** 2, axis=-1, keepdims=True)
    k = (k - mu) / jnp.sqrt(var + _EPS) * k_norm_w + k_norm_b

    posf = positions.astype(jnp.float32)
    inv_freq = 1.0 / (10000.0 ** (jnp.arange(0, _R, 2, dtype=jnp.float32) / _R))
    ang = posf[:, None] * inv_freq[None, :]
    cos, sin = jnp.cos(ang), jnp.sin(ang)
    ones = jnp.ones((_T, 64), jnp.float32)
    zeros = jnp.zeros((_T, 64), jnp.float32)
    c128 = jnp.concatenate([cos, cos, ones], axis=-1)
    s128 = jnp.concatenate([-sin, sin, zeros], axis=-1)

    bt = 256
    ngroups = _T // _GROUP
    keys = []
    for g in range(ngroups):
        r0, r1 = g * _GROUP, (g + 1) * _GROUP
        ncols = (g + 1) * _GROUP
        keys.append(pl.pallas_call(
            _make_logits_kernel(r0, ncols),
            grid=(_GROUP // bt,),
            in_specs=[
                pl.BlockSpec((bt, _H * _D), lambda i: (i, 0)),
                pl.BlockSpec((ncols, _D), lambda i: (0, 0)),
                pl.BlockSpec((bt, _DM), lambda i: (i, 0)),
                pl.BlockSpec((_DM, _H), lambda i: (0, 0)),
                pl.BlockSpec((bt, _D), lambda i: (i, 0)),
                pl.BlockSpec((bt, _D), lambda i: (i, 0)),
                pl.BlockSpec((ncols, _D), lambda i: (0, 0)),
                pl.BlockSpec((ncols, _D), lambda i: (0, 0)),
            ],
            out_specs=pl.BlockSpec((bt, ncols), lambda i: (i, 0)),
            out_shape=jax.ShapeDtypeStruct((_GROUP, ncols), jnp.uint32),
        )(q[r0:r1], k, hidden_states[r0:r1], w_proj, c128[r0:r1], s128[r0:r1],
          c128, s128))
    return _SC_TOPK(keys[0], keys[1], keys[2], keys[3])


# speedup vs baseline: 1.0899x; 1.0686x over previous
import functools

import jax
import jax.numpy as jnp
from jax import lax
from jax.experimental import pallas as pl
from jax.experimental.pallas import tpu as pltpu
from jax.experimental.pallas import tpu_sc as plsc

_T = 2048
_DM = 2048
_RQ = 1536
_H = 32
_D = 128
_R = 64
_TOPK = 512
_EPS = 1e-6


def _rope_apply(x, c128, s128, nheads):
    if nheads > 1:
        c = jnp.concatenate([c128] * nheads, axis=-1)
        s = jnp.concatenate([s128] * nheads, axis=-1)
    else:
        c, s = c128, s128
    lane = lax.broadcasted_iota(jnp.int32, x.shape, 1) % 128
    swapped = jnp.where(lane < 32, jnp.roll(x, -32, axis=1), jnp.roll(x, 32, axis=1))
    return x * c + swapped * s


def _logits_kernel(q_ref, k_ref, hs_ref, wproj_ref, c_ref, s_ref, ck_ref, sk_ref,
                   out_ref):
    bt = q_ref.shape[0]
    t0 = pl.program_id(0) * bt
    qr = _rope_apply(q_ref[...], c_ref[...], s_ref[...], _H)
    kr = _rope_apply(k_ref[...], ck_ref[...], sk_ref[...], 1)
    scale = (float(_D) ** -0.5) * (float(_H) ** -0.5)
    w = jax.lax.dot_general(hs_ref[...], wproj_ref[...], (((1,), (0,)), ((), ())),
                            preferred_element_type=jnp.float32) * scale
    for cc in range(_T // 512):
        c0 = cc * 512
        col = c0 + lax.broadcasted_iota(jnp.int32, (bt, 512), 1)
        masked = (jnp.int32(_T - 1) - col).astype(jnp.uint32)
        out_ref[:, c0:c0 + 512] = masked

        @pl.when(t0 + bt > c0)
        def _compute(cc=cc, c0=c0, col=col, masked=masked):
            krc = kr[c0:c0 + 512, :]
            acc = jnp.zeros((bt, 512), dtype=jnp.float32)
            for h in range(_H):
                qh = qr[:, h * _D:(h + 1) * _D]
                s = jax.lax.dot_general(qh, krc, (((1,), (1,)), ((), ())),
                                        preferred_element_type=jnp.float32)
                acc = acc + w[:, h:h + 1] * jnp.maximum(s, 0.0)
            row = t0 + lax.broadcasted_iota(jnp.int32, (bt, 512), 0)
            b = lax.bitcast_convert_type(acc, jnp.int32)
            m = (b >> 31) | jnp.int32(-2147483648)
            key = lax.bitcast_convert_type(b ^ m, jnp.uint32)
            out_ref[:, c0:c0 + 512] = jnp.where(row >= col, key, masked)


_NW = 32
_ROWS_PER = _T // _NW


def _sc_topk_kernel(keys_hbm, out_hbm, ka, ia, kb, ib, sem):
    wid = lax.axis_index("s") * 2 + lax.axis_index("c")
    iota16 = lax.iota(jnp.int32, 16)

    def vs(buf, i):
        return buf[pl.ds(i * 16, 16)]

    def st(buf, i, x):
        buf[pl.ds(i * 16, 16)] = x

    def bitonic_reg(ks, vi):
        nv = len(ks)
        D = nv // 2
        while D >= 1:
            for blk in range(0, nv, 2 * D):
                for j in range(D):
                    i0, i1 = blk + j, blk + j + D
                    cm = ks[i0] >= ks[i1]
                    hk = jnp.where(cm, ks[i0], ks[i1])
                    lk = jnp.where(cm, ks[i1], ks[i0])
                    hi = jnp.where(cm, vi[i0], vi[i1])
                    li = jnp.where(cm, vi[i1], vi[i0])
                    ks[i0], ks[i1] = hk, lk
                    vi[i0], vi[i1] = hi, li
            D //= 2
        for j in range(nv):
            ks[j], vi[j] = plsc.sort_key_val(ks[j], vi[j], descending=True)

    def merge_runs_reg(src_k, src_i, dst_k, dst_i, base, nv, top_only=False):
        ak = [vs(src_k, base + i) for i in range(nv)]
        ai = [vs(src_i, base + i) for i in range(nv)]
        bk = [lax.rev(vs(src_k, base + 2 * nv - 1 - i), (0,)) for i in range(nv)]
        bi = [lax.rev(vs(src_i, base + 2 * nv - 1 - i), (0,)) for i in range(nv)]
        uk, ui, vk_, vi_ = [], [], [], []
        for i in range(nv):
            cm = ak[i] >= bk[i]
            uk.append(jnp.where(cm, ak[i], bk[i]))
            ui.append(jnp.where(cm, ai[i], bi[i]))
            if not top_only:
                vk_.append(jnp.where(cm, bk[i], ak[i]))
                vi_.append(jnp.where(cm, bi[i], ai[i]))
        bitonic_reg(uk, ui)
        for i in range(nv):
            st(dst_k, base + i, uk[i])
            st(dst_i, base + i, ui[i])
        if not top_only:
            bitonic_reg(vk_, vi_)
            for i in range(nv):
                st(dst_k, base + nv + i, vk_[i])
                st(dst_i, base + nv + i, vi_[i])

    def merge_bitonic_mem(kbuf, ibuf, base, nv):
        D = nv // 2
        while D >= 4:
            for blk in range(0, nv, 2 * D):
                for j in range(D):
                    i0 = base + blk + j
                    i1 = i0 + D
                    a_k, b_k = vs(kbuf, i0), vs(kbuf, i1)
                    a_i, b_i = vs(ibuf, i0), vs(ibuf, i1)
                    cm = a_k >= b_k
                    st(kbuf, i0, jnp.where(cm, a_k, b_k))
                    st(kbuf, i1, jnp.where(cm, b_k, a_k))
                    st(ibuf, i0, jnp.where(cm, a_i, b_i))
                    st(ibuf, i1, jnp.where(cm, b_i, a_i))
            D //= 2
        for blk in range(0, nv, 8):
            ks = [vs(kbuf, base + blk + j) for j in range(8)]
            vi = [vs(ibuf, base + blk + j) for j in range(8)]
            bitonic_reg(ks, vi)
            for j in range(8):
                st(kbuf, base + blk + j, ks[j])
                st(ibuf, base + blk + j, vi[j])

    def build_chunk(c):
        base32 = c * 32

        def sort16(i, _):
            kk, vv = plsc.sort_key_val(vs(ka, base32 + i), iota16 + (base32 + i) * 16,
                                       descending=True)
            st(ka, base32 + i, kk)
            st(ia, base32 + i, vv)
            return 0

        lax.fori_loop(0, 32, sort16, 0, unroll=4)
        bufs = ((ka, ia), (kb, ib))
        cur = 0
        for nv in (1, 2, 4, 8, 16):
            src_k, src_i = bufs[cur]
            dst_k, dst_i = bufs[1 - cur]
            npairs = 32 // (2 * nv)

            def pair_body(p, _, src_k=src_k, src_i=src_i, dst_k=dst_k,
                          dst_i=dst_i, nv=nv, base32=base32):
                merge_runs_reg(src_k, src_i, dst_k, dst_i, base32 + p * 2 * nv, nv)
                return 0

            lax.fori_loop(0, npairs, pair_body, 0)
            cur = 1 - cur
        for i in range(32):
            st(ka, base32 + i, vs(kb, base32 + i))
            st(ia, base32 + i, vs(ib, base32 + i))

    def trunc_merge_into_top(c):
        for i in range(32):
            a_k, a_i = vs(kb, i), vs(ib, i)
            rj = c * 32 + 31 - i
            b_k = lax.rev(vs(ka, rj), (0,))
            b_i = lax.rev(vs(ia, rj), (0,))
            cm = a_k >= b_k
            st(kb, i, jnp.where(cm, a_k, b_k))
            st(ib, i, jnp.where(cm, a_i, b_i))
        merge_bitonic_mem(kb, ib, 0, 32)

    def row_body(r, _):
        row = wid + _NW * r
        nch = jnp.minimum(row // _TOPK + 1, jnp.int32(4))

        def chunk_body(c, _):
            pltpu.sync_copy(keys_hbm.at[row, pl.ds(c * _TOPK, _TOPK)],
                            ka.at[pl.ds(c * _TOPK, _TOPK)])
            build_chunk(c)
            return 0

        lax.fori_loop(0, nch, chunk_body, 0)
        for i in range(32):
            st(kb, i, vs(ka, i))
            st(ib, i, vs(ia, i))

        def merge_body(c, _):
            trunc_merge_into_top(c)
            return 0

        lax.fori_loop(1, nch, merge_body, 0)
        pltpu.sync_copy(ib.at[pl.ds(0, _TOPK)], out_hbm.at[row])
        return 0

    lax.fori_loop(0, _ROWS_PER, row_body, 0)


@functools.partial(
    pl.kernel,
    mesh=plsc.VectorSubcoreMesh(core_axis_name="c", subcore_axis_name="s"),
    out_type=jax.ShapeDtypeStruct((_T, _TOPK), jnp.int32),
    compiler_params=pltpu.CompilerParams(needs_layout_passes=False),
    scratch_types=[
        pltpu.VMEM((_T,), jnp.uint32),
        pltpu.VMEM((_T,), jnp.int32),
        pltpu.VMEM((_T,), jnp.uint32),
        pltpu.VMEM((_T,), jnp.int32),
        pltpu.SemaphoreType.DMA,
    ],
)
def _sc_topk(keys_hbm, out_hbm, ka, ia, kb, ib, sem):
    _sc_topk_kernel(keys_hbm, out_hbm, ka, ia, kb, ib, sem)


def kernel(hidden_states, q_lora, positions, wq_b, wk, k_norm_w, k_norm_b, w_proj):
    q = jnp.matmul(q_lora, wq_b)
    k = jnp.matmul(hidden_states, wk)
    mu = jnp.mean(k, axis=-1, keepdims=True)
    var = jnp.mean((k - mu) ** 2, axis=-1, keepdims=True)
    k = (k - mu) / jnp.sqrt(var + _EPS) * k_norm_w + k_norm_b

    posf = positions.astype(jnp.float32)
    inv_freq = 1.0 / (10000.0 ** (jnp.arange(0, _R, 2, dtype=jnp.float32) / _R))
    ang = posf[:, None] * inv_freq[None, :]
    cos, sin = jnp.cos(ang), jnp.sin(ang)
    ones = jnp.ones((_T, 64), jnp.float32)
    zeros = jnp.zeros((_T, 64), jnp.float32)
    c128 = jnp.concatenate([cos, cos, ones], axis=-1)
    s128 = jnp.concatenate([-sin, sin, zeros], axis=-1)

    bt = 256
    keys = pl.pallas_call(
        _logits_kernel,
        grid=(_T // bt,),
        in_specs=[
            pl.BlockSpec((bt, _H * _D), lambda i: (i, 0)),
            pl.BlockSpec((_T, _D), lambda i: (0, 0)),
            pl.BlockSpec((bt, _DM), lambda i: (i, 0)),
            pl.BlockSpec((_DM, _H), lambda i: (0, 0)),
            pl.BlockSpec((bt, _D), lambda i: (i, 0)),
            pl.BlockSpec((bt, _D), lambda i: (i, 0)),
            pl.BlockSpec((_T, _D), lambda i: (0, 0)),
            pl.BlockSpec((_T, _D), lambda i: (0, 0)),
        ],
        out_specs=pl.BlockSpec((bt, _T), lambda i: (i, 0)),
        out_shape=jax.ShapeDtypeStruct((_T, _T), jnp.uint32),
    )(q, k, hidden_states, w_proj, c128, s128, c128, s128)

    return _sc_topk(keys)
